# baseline (speedup 1.0000x reference)
import functools

import jax
import jax.numpy as jnp
from jax import lax
from jax.experimental import pallas as pl
from jax.experimental.pallas import tpu as pltpu
from jax.experimental.pallas import tpu_sc as plsc

N_NODES = 10000
N_EDGES = 320000
D = 128
DH = D // 2
NC = 2
NS = 16
EPS = N_EDGES // NS
CHUNK = 32
NCHUNK = EPS // CHUNK
NB = 5
NPASS = NCHUNK // NB
DW = DH // 2
ZROWS = 624
ZCH = 104
LANES = 16
FSL = DH // LANES

_mesh = plsc.VectorSubcoreMesh(core_axis_name="c", subcore_axis_name="s")


@functools.partial(
    pl.kernel,
    out_type=jax.ShapeDtypeStruct((NC, N_NODES, DH), jnp.float32),
    mesh=_mesh,
    compiler_params=pltpu.CompilerParams(
        needs_layout_passes=False, use_tc_tiling_on_sc=False),
    scratch_types=[
        pltpu.VMEM((EPS,), jnp.int32),
        pltpu.VMEM((NCHUNK, CHUNK), jnp.int32),
        pltpu.VMEM((EPS,), jnp.float32),
        pltpu.VMEM((NB, CHUNK, DW), jnp.int32),
        pltpu.VMEM((NB, CHUNK, DH), jnp.float32),
        pltpu.VMEM((ZCH, DH), jnp.float32),
        pltpu.VMEM_SHARED((N_NODES, DH), jnp.float32),
    ] + [pltpu.SemaphoreType.DMA] * (2 * NB),
)
def _sc_aggregate(x_hbm, src_hbm, dst_hbm, ew_hbm, out_hbm,
                  src_v, dst_v, w_v, raw_v, rows_v, zeros_v, acc, *sems):
    c = lax.axis_index("c")
    s = lax.axis_index("s")

    def _zb(i, carry):
        for j in range(FSL):
            zeros_v[i, pl.ds(j * LANES, LANES)] = jnp.zeros((LANES,), jnp.float32)
        return carry
    lax.fori_loop(0, ZCH, _zb, 0)
    r0 = s * ZROWS
    for k in range(ZROWS // ZCH):
        pltpu.sync_copy(zeros_v, acc.at[pl.ds(r0 + k * ZCH, ZCH)])

    @pl.when(s == NS - 1)
    def _zero_tail():
        pltpu.sync_copy(zeros_v.at[pl.ds(0, N_NODES - NS * ZROWS)],
                        acc.at[pl.ds(NS * ZROWS, N_NODES - NS * ZROWS)])
    plsc.subcore_barrier()

    pltpu.sync_copy(src_hbm.at[s], src_v)
    pltpu.sync_copy(dst_hbm.at[s], dst_v)
    pltpu.sync_copy(ew_hbm.at[s], w_v)

    cvec = jnp.full((LANES,), c, jnp.int32)

    def _xform(r, carry):
        sl = pl.ds(r * LANES, LANES)
        v = src_v[sl]
        src_v[sl] = v + v + cvec
        return carry
    lax.fori_loop(0, EPS // LANES, _xform, 0)

    sem_g = sems[:NB]
    sem_s = sems[NB:]

    shl16 = jnp.full((LANES,), 16, jnp.int32)
    hmask = jnp.full((LANES,), -65536, jnp.int32)

    def _scale(b, ci):
        w0 = w_v[pl.ds(ci * CHUNK, LANES)]
        w1 = w_v[pl.ds(ci * CHUNK + 16, LANES)]
        for e in range(CHUNK):
            if e < 16:
                w = jnp.full((LANES,), w0[e])
            else:
                w = jnp.full((LANES,), w1[e - 16])
            for j in range(DW // LANES):
                v = raw_v[b, e, pl.ds(j * LANES, LANES)]
                lo = plsc.bitcast(jnp.left_shift(v, shl16), jnp.float32)
                hi = plsc.bitcast(jnp.bitwise_and(v, hmask), jnp.float32)
                rows_v[b, e, pl.ds(j * LANES, LANES)] = lo * w
                rows_v[b, e, pl.ds(32 + j * LANES, LANES)] = hi * w

    def _pass(p, carry):
        for b in range(NB):
            ci = (p - 1) * NB + b

            @pl.when(p >= 1)
            def _proc():
                pltpu.make_async_copy(
                    x_hbm.at[src_v.at[pl.ds(ci * CHUNK, CHUNK)]],
                    raw_v.at[b], sem_g[b]).wait()

                @pl.when(p >= 2)
                def _wait_sc():
                    pltpu.make_async_copy(rows_v.at[b], acc.at[dst_v.at[0]],
                                          sem_s[b]).wait()
                _scale(b, ci)
                pltpu.async_copy(rows_v.at[b], acc.at[dst_v.at[ci]],
                                 sem_s[b], add=True)

        @pl.when(p <= NPASS - 1)
        def _refill():
            for b in range(NB):
                cn = p * NB + b
                pltpu.async_copy(
                    x_hbm.at[src_v.at[pl.ds(cn * CHUNK, CHUNK)]],
                    raw_v.at[b], sem_g[b])
        return carry
    lax.fori_loop(0, NPASS + 1, _pass, 0)

    for b in range(NB):
        pltpu.make_async_copy(rows_v.at[b], acc.at[dst_v.at[0]],
                              sem_s[b]).wait()

    plsc.subcore_barrier()

    pltpu.sync_copy(acc.at[pl.ds(r0, ZROWS)], out_hbm.at[c, pl.ds(r0, ZROWS)])

    @pl.when(s == NS - 1)
    def _pub_tail():
        pltpu.sync_copy(acc.at[pl.ds(NS * ZROWS, N_NODES - NS * ZROWS)],
                        out_hbm.at[c, pl.ds(NS * ZROWS, N_NODES - NS * ZROWS)])


_TC_BLK = 1000


def _tc_body(z_ref, w_ref, b_ref, o_ref):
    o_ref[...] = (
        jnp.dot(z_ref[0], w_ref[0], preferred_element_type=jnp.float32)
        + jnp.dot(z_ref[1], w_ref[1], preferred_element_type=jnp.float32)
        + b_ref[...]
    )


def _tc_combine(z, Wr, b2):
    return pl.pallas_call(
        _tc_body,
        grid=(N_NODES // _TC_BLK,),
        in_specs=[
            pl.BlockSpec((NC, _TC_BLK, DH), lambda i: (0, i, 0)),
            pl.BlockSpec((NC, DH, D), lambda i: (0, 0, 0)),
            pl.BlockSpec((1, D), lambda i: (0, 0)),
        ],
        out_specs=pl.BlockSpec((_TC_BLK, D), lambda i: (i, 0)),
        out_shape=jax.ShapeDtypeStruct((N_NODES, D), jnp.float32),
    )(z, Wr, b2)


def kernel(x, edge_index, edge_weight, W, b):
    dst_s = edge_index[0].astype(jnp.int32).reshape(NS, NCHUNK, CHUNK)
    src_s = edge_index[1].astype(jnp.int32).reshape(NS, EPS)
    ew = edge_weight.astype(jnp.float32).reshape(NS, EPS)
    xr = x.reshape(N_NODES, NC, 2, DW)
    lo = jax.lax.bitcast_convert_type(
        xr[:, :, 0, :].astype(jnp.bfloat16), jnp.uint16).astype(jnp.uint32)
    hi = jax.lax.bitcast_convert_type(
        xr[:, :, 1, :].astype(jnp.bfloat16), jnp.uint16).astype(jnp.uint32)
    xp = jax.lax.bitcast_convert_type(
        lo | (hi << jnp.uint32(16)), jnp.int32).reshape(N_NODES * NC, DW)
    z = _sc_aggregate(xp, src_s, dst_s, ew)
    return _tc_combine(z, W.reshape(NC, DH, D), b.reshape(1, D))


# speedup vs baseline: 1.0589x; 1.0589x over previous
import functools

import jax
import jax.numpy as jnp
from jax import lax
from jax.experimental import pallas as pl
from jax.experimental.pallas import tpu as pltpu
from jax.experimental.pallas import tpu_sc as plsc

N_NODES = 10000
N_EDGES = 320000
D = 128
DH = D // 2
NC = 2
NS = 16
EPS = N_EDGES // NS
CHUNK = 40
NCHUNK = EPS // CHUNK
NB = 5
NPASS = NCHUNK // NB
DW = DH // 2
ZROWS = 624
ZCH = 104
LANES = 16
FSL = DH // LANES

_mesh = plsc.VectorSubcoreMesh(core_axis_name="c", subcore_axis_name="s")


@functools.partial(
    pl.kernel,
    out_type=jax.ShapeDtypeStruct((NC, N_NODES, DH), jnp.float32),
    mesh=_mesh,
    compiler_params=pltpu.CompilerParams(
        needs_layout_passes=False, use_tc_tiling_on_sc=False),
    scratch_types=[
        pltpu.VMEM((EPS,), jnp.int32),
        pltpu.VMEM((NCHUNK, CHUNK), jnp.int32),
        pltpu.VMEM((EPS,), jnp.float32),
        pltpu.VMEM((NB, CHUNK, DW), jnp.int32),
        pltpu.VMEM((NB, CHUNK, DH), jnp.float32),
        pltpu.VMEM((ZCH, DH), jnp.float32),
        pltpu.VMEM_SHARED((N_NODES, DH), jnp.float32),
    ] + [pltpu.SemaphoreType.DMA] * (2 * NB),
)
def _sc_aggregate(x_hbm, src_hbm, dst_hbm, ew_hbm, out_hbm,
                  src_v, dst_v, w_v, raw_v, rows_v, zeros_v, acc, *sems):
    c = lax.axis_index("c")
    s = lax.axis_index("s")

    def _zb(i, carry):
        for j in range(FSL):
            zeros_v[i, pl.ds(j * LANES, LANES)] = jnp.zeros((LANES,), jnp.float32)
        return carry
    lax.fori_loop(0, ZCH, _zb, 0)
    r0 = s * ZROWS
    for k in range(ZROWS // ZCH):
        pltpu.sync_copy(zeros_v, acc.at[pl.ds(r0 + k * ZCH, ZCH)])

    @pl.when(s == NS - 1)
    def _zero_tail():
        pltpu.sync_copy(zeros_v.at[pl.ds(0, N_NODES - NS * ZROWS)],
                        acc.at[pl.ds(NS * ZROWS, N_NODES - NS * ZROWS)])
    plsc.subcore_barrier()

    pltpu.sync_copy(src_hbm.at[s], src_v)
    pltpu.sync_copy(dst_hbm.at[s], dst_v)
    pltpu.sync_copy(ew_hbm.at[s], w_v)

    cvec = jnp.full((LANES,), c, jnp.int32)

    def _xform(r, carry):
        sl = pl.ds(r * LANES, LANES)
        v = src_v[sl]
        src_v[sl] = v + v + cvec
        return carry
    lax.fori_loop(0, EPS // LANES, _xform, 0)

    sem_g = sems[:NB]
    sem_s = sems[NB:]

    shl16 = jnp.full((LANES,), 16, jnp.int32)
    hmask = jnp.full((LANES,), -65536, jnp.int32)

    def _scale(b, ci):
        w0 = w_v[pl.ds(ci * CHUNK, LANES)]
        w1 = w_v[pl.ds(ci * CHUNK + 16, LANES)]
        w2 = w_v[pl.ds(ci * CHUNK + 24, LANES)]
        for e in range(CHUNK):
            if e < 16:
                w = jnp.full((LANES,), w0[e])
            elif e < 32:
                w = jnp.full((LANES,), w1[e - 16])
            else:
                w = jnp.full((LANES,), w2[e - 24])
            for j in range(DW // LANES):
                v = raw_v[b, e, pl.ds(j * LANES, LANES)]
                lo = plsc.bitcast(jnp.left_shift(v, shl16), jnp.float32)
                hi = plsc.bitcast(jnp.bitwise_and(v, hmask), jnp.float32)
                rows_v[b, e, pl.ds(j * LANES, LANES)] = lo * w
                rows_v[b, e, pl.ds(32 + j * LANES, LANES)] = hi * w

    def _pass(p, carry):
        for b in range(NB):
            ci = (p - 1) * NB + b

            @pl.when(p >= 1)
            def _proc():
                pltpu.make_async_copy(
                    x_hbm.at[src_v.at[pl.ds(ci * CHUNK, CHUNK)]],
                    raw_v.at[b], sem_g[b]).wait()

                @pl.when(p >= 2)
                def _wait_sc():
                    pltpu.make_async_copy(rows_v.at[b], acc.at[dst_v.at[0]],
                                          sem_s[b]).wait()
                _scale(b, ci)
                pltpu.async_copy(rows_v.at[b], acc.at[dst_v.at[ci]],
                                 sem_s[b], add=True)

        @pl.when(p <= NPASS - 1)
        def _refill():
            for b in range(NB):
                cn = p * NB + b
                pltpu.async_copy(
                    x_hbm.at[src_v.at[pl.ds(cn * CHUNK, CHUNK)]],
                    raw_v.at[b], sem_g[b])
        return carry
    lax.fori_loop(0, NPASS + 1, _pass, 0)

    for b in range(NB):
        pltpu.make_async_copy(rows_v.at[b], acc.at[dst_v.at[0]],
                              sem_s[b]).wait()

    plsc.subcore_barrier()

    pltpu.sync_copy(acc.at[pl.ds(r0, ZROWS)], out_hbm.at[c, pl.ds(r0, ZROWS)])

    @pl.when(s == NS - 1)
    def _pub_tail():
        pltpu.sync_copy(acc.at[pl.ds(NS * ZROWS, N_NODES - NS * ZROWS)],
                        out_hbm.at[c, pl.ds(NS * ZROWS, N_NODES - NS * ZROWS)])


_TC_BLK = 1000


def _tc_body(z_ref, w_ref, b_ref, o_ref):
    o_ref[...] = (
        jnp.dot(z_ref[0], w_ref[0], preferred_element_type=jnp.float32)
        + jnp.dot(z_ref[1], w_ref[1], preferred_element_type=jnp.float32)
        + b_ref[...]
    )


def _tc_combine(z, Wr, b2):
    return pl.pallas_call(
        _tc_body,
        grid=(N_NODES // _TC_BLK,),
        in_specs=[
            pl.BlockSpec((NC, _TC_BLK, DH), lambda i: (0, i, 0)),
            pl.BlockSpec((NC, DH, D), lambda i: (0, 0, 0)),
            pl.BlockSpec((1, D), lambda i: (0, 0)),
        ],
        out_specs=pl.BlockSpec((_TC_BLK, D), lambda i: (i, 0)),
        out_shape=jax.ShapeDtypeStruct((N_NODES, D), jnp.float32),
    )(z, Wr, b2)


def kernel(x, edge_index, edge_weight, W, b):
    dst_s = edge_index[0].astype(jnp.int32).reshape(NS, NCHUNK, CHUNK)
    src_s = edge_index[1].astype(jnp.int32).reshape(NS, EPS)
    ew = edge_weight.astype(jnp.float32).reshape(NS, EPS)
    xr = x.reshape(N_NODES, NC, 2, DW)
    lo = jax.lax.bitcast_convert_type(
        xr[:, :, 0, :].astype(jnp.bfloat16), jnp.uint16).astype(jnp.uint32)
    hi = jax.lax.bitcast_convert_type(
        xr[:, :, 1, :].astype(jnp.bfloat16), jnp.uint16).astype(jnp.uint32)
    xp = jax.lax.bitcast_convert_type(
        lo | (hi << jnp.uint32(16)), jnp.int32).reshape(N_NODES * NC, DW)
    z = _sc_aggregate(xp, src_s, dst_s, ew)
    return _tc_combine(z, W.reshape(NC, DH, D), b.reshape(1, D))


# speedup vs baseline: 1.0691x; 1.0097x over previous
import functools

import jax
import jax.numpy as jnp
from jax import lax
from jax.experimental import pallas as pl
from jax.experimental.pallas import tpu as pltpu
from jax.experimental.pallas import tpu_sc as plsc

N_NODES = 10000
N_EDGES = 320000
D = 128
DH = D // 2
NC = 2
NS = 16
EPS = N_EDGES // NS
CHUNK = 40
NCHUNK = EPS // CHUNK
NB = 5
NPASS = NCHUNK // NB
ZROWS = 624
ZCH = 104
LANES = 16
FSL = DH // LANES

_mesh = plsc.VectorSubcoreMesh(core_axis_name="c", subcore_axis_name="s")


@functools.partial(
    pl.kernel,
    out_type=jax.ShapeDtypeStruct((NC, N_NODES, DH), jnp.float32),
    mesh=_mesh,
    compiler_params=pltpu.CompilerParams(
        needs_layout_passes=False, use_tc_tiling_on_sc=False),
    scratch_types=[
        pltpu.VMEM((EPS,), jnp.int32),
        pltpu.VMEM((NCHUNK, CHUNK), jnp.int32),
        pltpu.VMEM((EPS,), jnp.float32),
        pltpu.VMEM((NB, CHUNK, DH), jnp.float32),
        pltpu.VMEM((ZCH, DH), jnp.float32),
        pltpu.VMEM_SHARED((N_NODES, DH), jnp.float32),
    ] + [pltpu.SemaphoreType.DMA] * (2 * NB),
)
def _sc_aggregate(x_hbm, src_hbm, dst_hbm, ew_hbm, out_hbm,
                  src_v, dst_v, w_v, rows_v, zeros_v, acc, *sems):
    c = lax.axis_index("c")
    s = lax.axis_index("s")

    def _zb(i, carry):
        for j in range(FSL):
            zeros_v[i, pl.ds(j * LANES, LANES)] = jnp.zeros((LANES,), jnp.float32)
        return carry
    lax.fori_loop(0, ZCH, _zb, 0)
    r0 = s * ZROWS
    for k in range(ZROWS // ZCH):
        pltpu.sync_copy(zeros_v, acc.at[pl.ds(r0 + k * ZCH, ZCH)])

    @pl.when(s == NS - 1)
    def _zero_tail():
        pltpu.sync_copy(zeros_v.at[pl.ds(0, N_NODES - NS * ZROWS)],
                        acc.at[pl.ds(NS * ZROWS, N_NODES - NS * ZROWS)])
    plsc.subcore_barrier()

    pltpu.sync_copy(src_hbm.at[s], src_v)
    pltpu.sync_copy(dst_hbm.at[s], dst_v)
    pltpu.sync_copy(ew_hbm.at[s], w_v)

    cvec = jnp.full((LANES,), c, jnp.int32)

    def _xform(r, carry):
        sl = pl.ds(r * LANES, LANES)
        v = src_v[sl]
        src_v[sl] = v + v + cvec
        return carry
    lax.fori_loop(0, EPS // LANES, _xform, 0)

    sem_g = sems[:NB]
    sem_s = sems[NB:]

    def _scale(b, ci):
        w0 = w_v[pl.ds(ci * CHUNK, LANES)]
        w1 = w_v[pl.ds(ci * CHUNK + 16, LANES)]
        w2 = w_v[pl.ds(ci * CHUNK + 24, LANES)]
        for e in range(CHUNK):
            if e < 16:
                w = jnp.full((LANES,), w0[e])
            elif e < 32:
                w = jnp.full((LANES,), w1[e - 16])
            else:
                w = jnp.full((LANES,), w2[e - 24])
            for j in range(FSL):
                sl = pl.ds(j * LANES, LANES)
                rows_v[b, e, sl] = rows_v[b, e, sl] * w

    def _pass(p, carry):
        for b in range(NB):
            ci = (p - 1) * NB + b

            @pl.when(p >= 1)
            def _proc():
                pltpu.make_async_copy(
                    x_hbm.at[src_v.at[pl.ds(ci * CHUNK, CHUNK)]],
                    rows_v.at[b], sem_g[b]).wait()
                _scale(b, ci)
                pltpu.async_copy(rows_v.at[b], acc.at[dst_v.at[ci]],
                                 sem_s[b], add=True)

        @pl.when(p <= NPASS - 1)
        def _refill():
            for b in range(NB):
                cn = p * NB + b

                @pl.when(p >= 1)
                def _wait_sc():
                    pltpu.make_async_copy(rows_v.at[b], acc.at[dst_v.at[0]],
                                          sem_s[b]).wait()
                pltpu.async_copy(
                    x_hbm.at[src_v.at[pl.ds(cn * CHUNK, CHUNK)]],
                    rows_v.at[b], sem_g[b])
        return carry
    lax.fori_loop(0, NPASS + 1, _pass, 0)

    for b in range(NB):
        pltpu.make_async_copy(rows_v.at[b], acc.at[dst_v.at[0]],
                              sem_s[b]).wait()

    plsc.subcore_barrier()

    pltpu.sync_copy(acc.at[pl.ds(r0, ZROWS)], out_hbm.at[c, pl.ds(r0, ZROWS)])

    @pl.when(s == NS - 1)
    def _pub_tail():
        pltpu.sync_copy(acc.at[pl.ds(NS * ZROWS, N_NODES - NS * ZROWS)],
                        out_hbm.at[c, pl.ds(NS * ZROWS, N_NODES - NS * ZROWS)])


_TC_BLK = 1000


def _tc_body(z_ref, w_ref, b_ref, o_ref):
    o_ref[...] = (
        jnp.dot(z_ref[0], w_ref[0], preferred_element_type=jnp.float32)
        + jnp.dot(z_ref[1], w_ref[1], preferred_element_type=jnp.float32)
        + b_ref[...]
    )


def _tc_combine(z, Wr, b2):
    return pl.pallas_call(
        _tc_body,
        grid=(N_NODES // _TC_BLK,),
        in_specs=[
            pl.BlockSpec((NC, _TC_BLK, DH), lambda i: (0, i, 0)),
            pl.BlockSpec((NC, DH, D), lambda i: (0, 0, 0)),
            pl.BlockSpec((1, D), lambda i: (0, 0)),
        ],
        out_specs=pl.BlockSpec((_TC_BLK, D), lambda i: (i, 0)),
        out_shape=jax.ShapeDtypeStruct((N_NODES, D), jnp.float32),
    )(z, Wr, b2)


def kernel(x, edge_index, edge_weight, W, b):
    dst_s = edge_index[0].astype(jnp.int32).reshape(NS, NCHUNK, CHUNK)
    src_s = edge_index[1].astype(jnp.int32).reshape(NS, EPS)
    ew = edge_weight.astype(jnp.float32).reshape(NS, EPS)
    z = _sc_aggregate(x.reshape(N_NODES * 2, DH), src_s, dst_s, ew)
    return _tc_combine(z, W.reshape(NC, DH, D), b.reshape(1, D))
